# Initial kernel scaffold; baseline (speedup 1.0000x reference)
import jax
import jax.numpy as jnp
from jax.experimental import pallas as pl


def kernel(x, msgs_h, msgs_c, W_iou, b_iou, U_iou, b_uiou, W_f, b_wf, U_f, b_uf):
    raise NotImplementedError("write your pallas kernel here")


# speedup vs baseline: 1.6134x; 1.6134x over previous
import functools

import jax
import jax.numpy as jnp
from jax.experimental import pallas as pl


def _tree_lstm_block(x_ref, mh_ref, mc_ref,
                     Wf_t_ref, bwf_ref, Uf_t_ref, buf_ref,
                     Wiou_t_ref, biou_ref, Uiou_t_ref, buiou_ref,
                     h_ref, c_ref, *, block_n: int, k: int, h_dim: int):
    xb = x_ref[...]
    mh = mh_ref[...]
    mc = mc_ref[...]

    h_tild = jnp.sum(mh, axis=1)

    wx = jnp.dot(xb, Wf_t_ref[...], preferred_element_type=jnp.float32)
    wx = wx + bwf_ref[...] + buf_ref[...]

    mh2 = mh.reshape(block_n * k, h_dim)
    uh = jnp.dot(mh2, Uf_t_ref[...], preferred_element_type=jnp.float32)
    f = jax.nn.sigmoid(uh.reshape(block_n, k, h_dim) + wx[:, None, :])
    c_tild = jnp.sum(f * mc, axis=1)

    iou = (jnp.dot(xb, Wiou_t_ref[...], preferred_element_type=jnp.float32)
           + jnp.dot(h_tild, Uiou_t_ref[...],
                     preferred_element_type=jnp.float32)
           + biou_ref[...] + buiou_ref[...])
    i_g = jax.nn.sigmoid(iou[:, :h_dim])
    o_g = jax.nn.sigmoid(iou[:, h_dim:2 * h_dim])
    u_g = jnp.tanh(iou[:, 2 * h_dim:])

    c = i_g * u_g + c_tild
    h_ref[...] = o_g * jnp.tanh(c)
    c_ref[...] = c


def kernel(x, msgs_h, msgs_c, W_iou, b_iou, U_iou, b_uiou, W_f, b_wf, U_f, b_uf):
    n, k, h_dim = msgs_h.shape
    x_dim = x.shape[1]

    block_n = 400
    assert n % block_n == 0
    grid = (n // block_n,)

    full = lambda i: (0, 0)
    body = functools.partial(_tree_lstm_block, block_n=block_n, k=k,
                             h_dim=h_dim)

    h, c = pl.pallas_call(
        body,
        grid=grid,
        in_specs=[
            pl.BlockSpec((block_n, x_dim), lambda i: (i, 0)),
            pl.BlockSpec((block_n, k, h_dim), lambda i: (i, 0, 0)),
            pl.BlockSpec((block_n, k, h_dim), lambda i: (i, 0, 0)),
            pl.BlockSpec((x_dim, h_dim), full),
            pl.BlockSpec((1, h_dim), full),
            pl.BlockSpec((h_dim, h_dim), full),
            pl.BlockSpec((1, h_dim), full),
            pl.BlockSpec((x_dim, 3 * h_dim), full),
            pl.BlockSpec((1, 3 * h_dim), full),
            pl.BlockSpec((h_dim, 3 * h_dim), full),
            pl.BlockSpec((1, 3 * h_dim), full),
        ],
        out_specs=[
            pl.BlockSpec((block_n, h_dim), lambda i: (i, 0)),
            pl.BlockSpec((block_n, h_dim), lambda i: (i, 0)),
        ],
        out_shape=[
            jax.ShapeDtypeStruct((n, h_dim), jnp.float32),
            jax.ShapeDtypeStruct((n, h_dim), jnp.float32),
        ],
    )(
        x, msgs_h, msgs_c,
        W_f.T, b_wf.reshape(1, h_dim),
        U_f.T, b_uf.reshape(1, h_dim),
        W_iou.T, b_iou.reshape(1, 3 * h_dim),
        U_iou.T, b_uiou.reshape(1, 3 * h_dim),
    )
    return (h, c)
